# Initial kernel scaffold; baseline (speedup 1.0000x reference)
import jax
import jax.numpy as jnp
from jax.experimental import pallas as pl


def kernel(local_features, ch_idxs, emb_table):
    raise NotImplementedError("write your pallas kernel here")


# speedup vs baseline: 3.4803x; 3.4803x over previous
import functools
import math

import jax
import jax.numpy as jnp
from jax.experimental import pallas as pl
from jax.experimental.pallas import tpu as pltpu

SPAT_DIM = 192
TIME_DIM = 192
MAX_N_TIMES = int(600.0 * 4.0)


def _time_table(n_times, n_dim, max_n_times):
    position = jnp.arange(n_times, dtype=jnp.float32)[:, None]
    div = jnp.exp(
        jnp.arange(0, n_dim, 2, dtype=jnp.float32) * (-math.log(max_n_times) / n_dim)
    )
    ang = position * div
    return jnp.stack([jnp.sin(ang), jnp.cos(ang)], axis=-1).reshape(n_times, n_dim)


def _body(idx_ref, emb_ref, tt_ref, out_ref, *, n_chans, n_times):
    b = pl.program_id(0)
    tt = tt_ref[...]
    for c in range(n_chans):
        row = emb_ref[idx_ref[b, c], :]
        out_ref[0, c * n_times:(c + 1) * n_times, :] = row[None, :] + tt


def kernel(local_features, ch_idxs, emb_table):
    B, n_chans_times, emb_dim = local_features.shape
    n_chans = ch_idxs.shape[1]
    n_times = n_chans_times // n_chans

    tt = _time_table(n_times, TIME_DIM, MAX_N_TIMES).astype(local_features.dtype)
    tt_pad = jnp.pad(tt, ((0, 0), (SPAT_DIM, 0)))
    emb_pad = jnp.pad(emb_table, ((0, 0), (0, TIME_DIM)))
    n_rows = emb_pad.shape[0]

    body = functools.partial(_body, n_chans=n_chans, n_times=n_times)
    return pl.pallas_call(
        body,
        grid_spec=pltpu.PrefetchScalarGridSpec(
            num_scalar_prefetch=1,
            grid=(B,),
            in_specs=[
                pl.BlockSpec((n_rows, emb_dim), lambda b, idx: (0, 0)),
                pl.BlockSpec((n_times, emb_dim), lambda b, idx: (0, 0)),
            ],
            out_specs=pl.BlockSpec((1, n_chans_times, emb_dim),
                                   lambda b, idx: (b, 0, 0)),
        ),
        out_shape=jax.ShapeDtypeStruct((B, n_chans_times, emb_dim),
                                       local_features.dtype),
    )(ch_idxs, emb_pad, tt_pad)
